# baseline (speedup 1.0000x reference)
import functools

import jax
import jax.numpy as jnp
from jax import lax
from jax.experimental import pallas as pl
from jax.experimental.pallas import tpu as pltpu
from jax.experimental.pallas import tpu_sc as plsc

N_NODES = 10000
D_FEAT = 128
NUM_GRAPHS = 64
NC = 2
NS = 16
NW = NC * NS
CHUNK = 320
TAIL = N_NODES - (NW - 1) * CHUNK
N_PAD = 10240


def _sc_body(x_hbm, b_hbm, out_sum, xb, ixb, ixb2, ixt, zrow, shared_sum,
             sem_x, sem_i, sem_s):
    c = lax.axis_index("c")
    s = lax.axis_index("s")
    wid = s * NC + c
    base = wid * CHUNK
    last = wid == NW - 1
    not_last = jnp.logical_not(last)

    half = CHUNK // 2

    @pl.when(not_last)
    def _():
        pltpu.async_copy(x_hbm.at[pl.ds(base, half)],
                         xb.at[pl.ds(0, half)], sem_x)
        pltpu.async_copy(x_hbm.at[pl.ds(base + half, half)],
                         xb.at[pl.ds(half, half)], sem_x)
        pltpu.async_copy(b_hbm.at[pl.ds(base, half)], ixb, sem_i)
        pltpu.async_copy(b_hbm.at[pl.ds(base + half, half)], ixb2, sem_i)

    @pl.when(last)
    def _():
        pltpu.async_copy(x_hbm.at[pl.ds(base, TAIL)],
                         xb.at[pl.ds(0, TAIL)], sem_x)
        pltpu.async_copy(b_hbm.at[pl.ds(base, TAIL)], ixt, sem_i)

    zvec = jnp.zeros((16,), jnp.float32)
    for r in range(4):
        for j in range(D_FEAT // 16):
            zrow[r, pl.ds(j * 16, 16)] = zvec

    pltpu.sync_copy(zrow, shared_sum.at[pl.ds(s * 4, 4)])
    plsc.subcore_barrier()

    @pl.when(not_last)
    def _():
        pltpu.make_async_copy(x_hbm.at[pl.ds(base, half)],
                              xb.at[pl.ds(0, half)], sem_x).wait()
        pltpu.make_async_copy(x_hbm.at[pl.ds(base + half, half)],
                              xb.at[pl.ds(half, half)], sem_x).wait()
        pltpu.make_async_copy(b_hbm.at[pl.ds(base, half)], ixb, sem_i).wait()
        pltpu.make_async_copy(b_hbm.at[pl.ds(base + half, half)], ixb2,
                              sem_i).wait()
        d0 = pltpu.async_copy(xb.at[pl.ds(0, half)], shared_sum.at[ixb],
                              sem_s, add=True)
        d1 = pltpu.async_copy(xb.at[pl.ds(half, half)], shared_sum.at[ixb2],
                              sem_s, add=True)
        d0.wait()
        d1.wait()

    @pl.when(last)
    def _():
        pltpu.make_async_copy(x_hbm.at[pl.ds(base, TAIL)],
                              xb.at[pl.ds(0, TAIL)], sem_x).wait()
        pltpu.make_async_copy(b_hbm.at[pl.ds(base, TAIL)], ixt, sem_i).wait()
        pltpu.async_copy(xb.at[pl.ds(0, TAIL)], shared_sum.at[ixt],
                         sem_s, add=True).wait()

    plsc.subcore_barrier()

    pltpu.sync_copy(shared_sum.at[pl.ds(s * 4, 4)],
                    out_sum.at[c, pl.ds(s * 4, 4), :])


_sc_segment_sum = functools.partial(
    pl.kernel,
    out_type=jax.ShapeDtypeStruct((NC, NUM_GRAPHS, D_FEAT), jnp.float32),
    mesh=plsc.VectorSubcoreMesh(core_axis_name="c", subcore_axis_name="s"),
    scratch_types=[
        pltpu.VMEM((CHUNK, D_FEAT), jnp.float32),
        pltpu.VMEM((CHUNK // 2,), jnp.int32),
        pltpu.VMEM((CHUNK // 2,), jnp.int32),
        pltpu.VMEM((TAIL,), jnp.int32),
        pltpu.VMEM((4, D_FEAT), jnp.float32),
        pltpu.VMEM_SHARED((NUM_GRAPHS, D_FEAT), jnp.float32),
        pltpu.SemaphoreType.DMA,
        pltpu.SemaphoreType.DMA,
        pltpu.SemaphoreType.DMA,
    ],
)(_sc_body)


def _mlp_body(sum_ref, ids_ref, u_ref, w1_ref, b1_ref, w2_ref, b2_ref, o_ref):
    sums = sum_ref[0] + sum_ref[1]
    ids = ids_ref[...]
    gid = lax.broadcasted_iota(jnp.int32, (NUM_GRAPHS, 1), 0)
    onehot = (gid == ids).astype(jnp.float32)
    counts = jnp.sum(onehot, axis=1, keepdims=True)
    mean = sums / jnp.maximum(counts, 1.0)
    u = u_ref[...]
    h = (
        jnp.dot(u, w1_ref[0:64, :], preferred_element_type=jnp.float32)
        + jnp.dot(mean, w1_ref[64:192, :], preferred_element_type=jnp.float32)
        + b1_ref[...]
    )
    h = jnp.maximum(h, 0.0)
    o_ref[...] = (
        jnp.dot(h, w2_ref[...], preferred_element_type=jnp.float32) + b2_ref[...]
    )


def _tc_mlp(sums_p, ids_row, u, W1, b1, W2, b2):
    return pl.pallas_call(
        _mlp_body,
        out_shape=jax.ShapeDtypeStruct((u.shape[0], W2.shape[1]), jnp.float32),
    )(sums_p, ids_row, u, W1, b1, W2, b2)


def kernel(x, edge_index, edge_attr, u, batch, W1, b1, W2, b2):
    sums_p = _sc_segment_sum(x, batch)
    ids_row = jnp.concatenate(
        [batch, jnp.full((N_PAD - N_NODES,), NUM_GRAPHS, jnp.int32)]
    ).reshape(1, N_PAD)
    return _tc_mlp(
        sums_p, ids_row, u, W1, b1.reshape(1, -1), W2, b2.reshape(1, -1)
    )


# speedup vs baseline: 1.0071x; 1.0071x over previous
import functools

import jax
import jax.numpy as jnp
from jax import lax
from jax.experimental import pallas as pl
from jax.experimental.pallas import tpu as pltpu
from jax.experimental.pallas import tpu_sc as plsc

N_NODES = 10000
D_FEAT = 128
NUM_GRAPHS = 64
NC = 2
NS = 16
NW = NC * NS
CHUNK = 320
TAIL = N_NODES - (NW - 1) * CHUNK
N_PAD = 10240


def _sc_body(x_hbm, b_hbm, out_sum, xb, ixb, ixb2, ixb3, ixt, zrow,
             shared_sum, sem_x, sem_i, sem_s):
    c = lax.axis_index("c")
    s = lax.axis_index("s")
    wid = s * NC + c
    base = wid * CHUNK
    last = wid == NW - 1
    not_last = jnp.logical_not(last)

    q = CHUNK // 4
    ixs = [ixb, ixb2, ixb3, ixt]

    pltpu.async_copy(x_hbm.at[pl.ds(base, q)], xb.at[pl.ds(0, q)], sem_x)
    pltpu.async_copy(b_hbm.at[pl.ds(base, q)], ixs[0], sem_i)

    @pl.when(not_last)
    def _():
        for k in range(1, 4):
            pltpu.async_copy(x_hbm.at[pl.ds(base + k * q, q)],
                             xb.at[pl.ds(k * q, q)], sem_x)
            pltpu.async_copy(b_hbm.at[pl.ds(base + k * q, q)], ixs[k], sem_i)

    zvec = jnp.zeros((16,), jnp.float32)
    for r in range(4):
        for j in range(D_FEAT // 16):
            zrow[r, pl.ds(j * 16, 16)] = zvec

    pltpu.sync_copy(zrow, shared_sum.at[pl.ds(s * 4, 4)])
    plsc.subcore_barrier()

    def _drain(k):
        pltpu.make_async_copy(x_hbm.at[pl.ds(base + k * q, q)],
                              xb.at[pl.ds(k * q, q)], sem_x).wait()
        pltpu.make_async_copy(b_hbm.at[pl.ds(base + k * q, q)], ixs[k],
                              sem_i).wait()

    def _scat(k):
        return pltpu.async_copy(xb.at[pl.ds(k * q, q)],
                                shared_sum.at[ixs[k]], sem_s, add=True)

    _drain(0)

    @pl.when(not_last)
    def _():
        for k in range(1, 4):
            _drain(k)
        ds_ = [_scat(k) for k in range(4)]
        for d in ds_:
            d.wait()

    @pl.when(last)
    def _():
        _scat(0).wait()

    plsc.subcore_barrier()

    @pl.when(s == 0)
    def _():
        pltpu.sync_copy(shared_sum, out_sum.at[c])


_sc_segment_sum = functools.partial(
    pl.kernel,
    out_type=jax.ShapeDtypeStruct((NC, NUM_GRAPHS, D_FEAT), jnp.float32),
    mesh=plsc.VectorSubcoreMesh(core_axis_name="c", subcore_axis_name="s"),
    scratch_types=[
        pltpu.VMEM((CHUNK, D_FEAT), jnp.float32),
        pltpu.VMEM((CHUNK // 4,), jnp.int32),
        pltpu.VMEM((CHUNK // 4,), jnp.int32),
        pltpu.VMEM((CHUNK // 4,), jnp.int32),
        pltpu.VMEM((TAIL,), jnp.int32),
        pltpu.VMEM((4, D_FEAT), jnp.float32),
        pltpu.VMEM_SHARED((NUM_GRAPHS, D_FEAT), jnp.float32),
        pltpu.SemaphoreType.DMA,
        pltpu.SemaphoreType.DMA,
        pltpu.SemaphoreType.DMA,
    ],
)(_sc_body)


def _mlp_body(sum_ref, ids_ref, u_ref, w1_ref, b1_ref, w2_ref, b2_ref, o_ref):
    sums = sum_ref[0] + sum_ref[1]
    ids = ids_ref[...]
    gid = lax.broadcasted_iota(jnp.int32, (NUM_GRAPHS, 1), 0)
    onehot = (gid == ids).astype(jnp.float32)
    counts = jnp.sum(onehot, axis=1, keepdims=True)
    mean = sums / jnp.maximum(counts, 1.0)
    u = u_ref[...]
    h = (
        jnp.dot(u, w1_ref[0:64, :], preferred_element_type=jnp.float32)
        + jnp.dot(mean, w1_ref[64:192, :], preferred_element_type=jnp.float32)
        + b1_ref[...]
    )
    h = jnp.maximum(h, 0.0)
    o_ref[...] = (
        jnp.dot(h, w2_ref[...], preferred_element_type=jnp.float32) + b2_ref[...]
    )


def _tc_mlp(sums_p, ids_row, u, W1, b1, W2, b2):
    return pl.pallas_call(
        _mlp_body,
        out_shape=jax.ShapeDtypeStruct((u.shape[0], W2.shape[1]), jnp.float32),
    )(sums_p, ids_row, u, W1, b1, W2, b2)


def kernel(x, edge_index, edge_attr, u, batch, W1, b1, W2, b2):
    sums_p = _sc_segment_sum(x, batch)
    ids_row = jnp.concatenate(
        [batch, jnp.full((N_PAD - N_NODES,), NUM_GRAPHS, jnp.int32)]
    ).reshape(1, N_PAD)
    return _tc_mlp(
        sums_p, ids_row, u, W1, b1.reshape(1, -1), W2, b2.reshape(1, -1)
    )


# speedup vs baseline: 1.0210x; 1.0138x over previous
import functools

import jax
import jax.numpy as jnp
from jax import lax
from jax.experimental import pallas as pl
from jax.experimental.pallas import tpu as pltpu
from jax.experimental.pallas import tpu_sc as plsc

N_NODES = 10000
D_FEAT = 128
NUM_GRAPHS = 64
NC = 2
NS = 16
NW = NC * NS
CHUNK = 320
TAIL = N_NODES - (NW - 1) * CHUNK
N_PAD = 10240


def _sc_body(x_hbm, b_hbm, out_sum, xb, ixb, ixb2, ixb3, ixt, zrow,
             shared_sum, sem_x, sem_i, sem_s):
    c = lax.axis_index("c")
    s = lax.axis_index("s")
    wid = s * NC + c
    base = wid * CHUNK
    last = wid == NW - 1
    not_last = jnp.logical_not(last)

    q = CHUNK // 4
    ixs = [ixb, ixb2, ixb3, ixt]

    pltpu.async_copy(x_hbm.at[pl.ds(base, q)], xb.at[pl.ds(0, q)], sem_x)
    pltpu.async_copy(b_hbm.at[pl.ds(base, q)], ixs[0], sem_i)

    @pl.when(not_last)
    def _():
        for k in range(1, 4):
            pltpu.async_copy(x_hbm.at[pl.ds(base + k * q, q)],
                             xb.at[pl.ds(k * q, q)], sem_x)
            pltpu.async_copy(b_hbm.at[pl.ds(base + k * q, q)], ixs[k], sem_i)

    zvec = jnp.zeros((16,), jnp.float32)
    for r in range(4):
        for j in range(D_FEAT // 16):
            zrow[r, pl.ds(j * 16, 16)] = zvec

    pltpu.sync_copy(zrow, shared_sum.at[pl.ds(s * 4, 4)])
    plsc.subcore_barrier()

    def _drain(k):
        pltpu.make_async_copy(x_hbm.at[pl.ds(base + k * q, q)],
                              xb.at[pl.ds(k * q, q)], sem_x).wait()
        pltpu.make_async_copy(b_hbm.at[pl.ds(base + k * q, q)], ixs[k],
                              sem_i).wait()

    def _scat(k):
        return pltpu.async_copy(xb.at[pl.ds(k * q, q)],
                                shared_sum.at[ixs[k]], sem_s, add=True)

    _drain(0)

    @pl.when(not_last)
    def _():
        ds_ = [_scat(0)]
        for k in range(1, 4):
            _drain(k)
            ds_.append(_scat(k))
        for d in ds_:
            d.wait()

    @pl.when(last)
    def _():
        _scat(0).wait()

    plsc.subcore_barrier()

    pltpu.sync_copy(shared_sum.at[pl.ds(s * 4, 4)],
                    out_sum.at[c, pl.ds(s * 4, 4)])


_sc_segment_sum = functools.partial(
    pl.kernel,
    out_type=jax.ShapeDtypeStruct((NC, NUM_GRAPHS, D_FEAT), jnp.float32),
    mesh=plsc.VectorSubcoreMesh(core_axis_name="c", subcore_axis_name="s"),
    scratch_types=[
        pltpu.VMEM((CHUNK, D_FEAT), jnp.float32),
        pltpu.VMEM((CHUNK // 4,), jnp.int32),
        pltpu.VMEM((CHUNK // 4,), jnp.int32),
        pltpu.VMEM((CHUNK // 4,), jnp.int32),
        pltpu.VMEM((TAIL,), jnp.int32),
        pltpu.VMEM((4, D_FEAT), jnp.float32),
        pltpu.VMEM_SHARED((NUM_GRAPHS, D_FEAT), jnp.float32),
        pltpu.SemaphoreType.DMA,
        pltpu.SemaphoreType.DMA,
        pltpu.SemaphoreType.DMA,
    ],
)(_sc_body)


def _mlp_body(sum_ref, ids_ref, u_ref, w1_ref, b1_ref, w2_ref, b2_ref, o_ref):
    sums = sum_ref[0] + sum_ref[1]
    ids = ids_ref[...]
    gid = lax.broadcasted_iota(jnp.int32, (NUM_GRAPHS, 1), 0)
    onehot = (gid == ids).astype(jnp.float32)
    counts = jnp.sum(onehot, axis=1, keepdims=True)
    mean = sums / jnp.maximum(counts, 1.0)
    u = u_ref[...]
    h = (
        jnp.dot(u, w1_ref[0:64, :], preferred_element_type=jnp.float32)
        + jnp.dot(mean, w1_ref[64:192, :], preferred_element_type=jnp.float32)
        + b1_ref[...]
    )
    h = jnp.maximum(h, 0.0)
    o_ref[...] = (
        jnp.dot(h, w2_ref[...], preferred_element_type=jnp.float32) + b2_ref[...]
    )


def _tc_mlp(sums_p, ids_row, u, W1, b1, W2, b2):
    return pl.pallas_call(
        _mlp_body,
        out_shape=jax.ShapeDtypeStruct((u.shape[0], W2.shape[1]), jnp.float32),
    )(sums_p, ids_row, u, W1, b1, W2, b2)


def kernel(x, edge_index, edge_attr, u, batch, W1, b1, W2, b2):
    sums_p = _sc_segment_sum(x, batch)
    ids_row = jnp.concatenate(
        [batch, jnp.full((N_PAD - N_NODES,), NUM_GRAPHS, jnp.int32)]
    ).reshape(1, N_PAD)
    return _tc_mlp(
        sums_p, ids_row, u, W1, b1.reshape(1, -1), W2, b2.reshape(1, -1)
    )


# speedup vs baseline: 1.0307x; 1.0095x over previous
import functools

import jax
import jax.numpy as jnp
from jax import lax
from jax.experimental import pallas as pl
from jax.experimental.pallas import tpu as pltpu
from jax.experimental.pallas import tpu_sc as plsc

N_NODES = 10000
D_FEAT = 128
NUM_GRAPHS = 64
NC = 2
NS = 16
NW = NC * NS
CHUNK = 320
TAIL = N_NODES - (NW - 1) * CHUNK
N_PAD = 10240


def _sc_body(x_hbm, b_hbm, out_sum, xb, ixb, ixb2, ixb3, ixt, zrow,
             shared_sum, sem_x, sem_i, sem_s):
    c = lax.axis_index("c")
    s = lax.axis_index("s")
    wid = s * NC + c
    base = wid * CHUNK
    last = wid == NW - 1
    not_last = jnp.logical_not(last)

    q = CHUNK // 4
    ixs = [ixb, ixb2, ixb3, ixt]

    pltpu.async_copy(x_hbm.at[pl.ds(base, q)], xb.at[pl.ds(0, q)], sem_x)
    pltpu.async_copy(b_hbm.at[pl.ds(base, q)], ixs[0], sem_i)

    @pl.when(not_last)
    def _():
        for k in range(1, 4):
            pltpu.async_copy(x_hbm.at[pl.ds(base + k * q, q)],
                             xb.at[pl.ds(k * q, q)], sem_x)
            pltpu.async_copy(b_hbm.at[pl.ds(base + k * q, q)], ixs[k], sem_i)

    zvec = jnp.zeros((16,), jnp.float32)
    for r in range(4):
        for j in range(D_FEAT // 16):
            zrow[r, pl.ds(j * 16, 16)] = zvec

    pltpu.sync_copy(zrow, shared_sum.at[pl.ds(s * 4, 4)])
    plsc.subcore_barrier()

    def _drain(k):
        pltpu.make_async_copy(x_hbm.at[pl.ds(base + k * q, q)],
                              xb.at[pl.ds(k * q, q)], sem_x).wait()
        pltpu.make_async_copy(b_hbm.at[pl.ds(base + k * q, q)], ixs[k],
                              sem_i).wait()

    def _scat(k):
        return pltpu.async_copy(xb.at[pl.ds(k * q, q)],
                                shared_sum.at[ixs[k]], sem_s, add=True)

    _drain(0)

    @pl.when(not_last)
    def _():
        ds_ = [_scat(0)]
        for k in range(1, 4):
            _drain(k)
            ds_.append(_scat(k))
        for d in ds_:
            d.wait()

    @pl.when(last)
    def _():
        _scat(0).wait()

    plsc.subcore_barrier()

    pltpu.sync_copy(shared_sum.at[pl.ds(s * 4, 4)],
                    out_sum.at[c, pl.ds(s * 4, 4)])


_sc_segment_sum = functools.partial(
    pl.kernel,
    out_type=jax.ShapeDtypeStruct((NC, NUM_GRAPHS, D_FEAT), jnp.float32),
    mesh=plsc.VectorSubcoreMesh(core_axis_name="c", subcore_axis_name="s"),
    scratch_types=[
        pltpu.VMEM((CHUNK, D_FEAT), jnp.float32),
        pltpu.VMEM((CHUNK // 4,), jnp.int32),
        pltpu.VMEM((CHUNK // 4,), jnp.int32),
        pltpu.VMEM((CHUNK // 4,), jnp.int32),
        pltpu.VMEM((TAIL,), jnp.int32),
        pltpu.VMEM((4, D_FEAT), jnp.float32),
        pltpu.VMEM_SHARED((NUM_GRAPHS, D_FEAT), jnp.float32),
        pltpu.SemaphoreType.DMA,
        pltpu.SemaphoreType.DMA,
        pltpu.SemaphoreType.DMA,
    ],
)(_sc_body)


def _mlp_body(sum_ref, ids_ref, u_ref, w1_ref, b1_ref, w2_ref, b2_ref, o_ref):
    sums = sum_ref[0] + sum_ref[1]
    ids = ids_ref[...]
    gid = lax.broadcasted_iota(jnp.int32, (NUM_GRAPHS, 1), 0)
    onehot = (gid == ids).astype(jnp.float32)
    counts = jnp.sum(onehot, axis=1, keepdims=True)
    mean = sums / jnp.maximum(counts, 1.0)
    u = u_ref[...]
    h = (
        jnp.dot(u, w1_ref[0:64, :], preferred_element_type=jnp.float32)
        + jnp.dot(mean, w1_ref[64:192, :], preferred_element_type=jnp.float32)
        + b1_ref[...]
    )
    h = jnp.maximum(h, 0.0)
    o_ref[...] = (
        jnp.dot(h, w2_ref[...], preferred_element_type=jnp.float32) + b2_ref[...]
    )


def _tc_mlp(sums_p, ids_row, u, W1, b1, W2, b2):
    return pl.pallas_call(
        _mlp_body,
        out_shape=jax.ShapeDtypeStruct((u.shape[0], W2.shape[1]), jnp.float32),
    )(sums_p, ids_row, u, W1, b1, W2, b2)


def kernel(x, edge_index, edge_attr, u, batch, W1, b1, W2, b2):
    sums_p = _sc_segment_sum(x, batch)
    return _tc_mlp(
        sums_p, batch.reshape(1, N_NODES), u, W1, b1.reshape(1, -1),
        W2, b2.reshape(1, -1)
    )
